# baseline (speedup 1.0000x reference)
import functools

import jax
import jax.numpy as jnp
from jax.experimental import pallas as pl
from jax.experimental.pallas import tpu as pltpu

_BM = 200


def _body(x_ref, l_ref, u_ref, wi_ref, ws_ref, out_ref, xb_ref):
    m = pl.program_id(0)

    @pl.when(m == 0)
    def _cast_x_once():
        xb_ref[...] = x_ref[...].astype(jnp.bfloat16)

    xb = xb_ref[...]
    lb = l_ref[...].astype(jnp.bfloat16)
    ub = u_ref[...].astype(jnp.bfloat16)
    t_l = jnp.dot(lb, xb, preferred_element_type=jnp.float32)
    t_u = jnp.dot(ub, xb, preferred_element_type=jnp.float32)
    t = (jnp.dot(t_l, wi_ref[...], preferred_element_type=jnp.float32)
         + jnp.dot(t_u, ws_ref[...], preferred_element_type=jnp.float32))
    out_ref[...] = jnp.maximum(t, 0.0)


def _run(x, lower, upper, w_irr, w_sol, bm):
    n, d = x.shape
    d_out = w_irr.shape[1]
    return pl.pallas_call(
        _body,
        grid=(n // bm,),
        in_specs=[
            pl.BlockSpec((n, d), lambda m: (0, 0)),
            pl.BlockSpec((bm, n), lambda m: (m, 0)),
            pl.BlockSpec((bm, n), lambda m: (m, 0)),
            pl.BlockSpec((d, d_out), lambda m: (0, 0)),
            pl.BlockSpec((d, d_out), lambda m: (0, 0)),
        ],
        out_specs=pl.BlockSpec((bm, d_out), lambda m: (m, 0)),
        out_shape=jax.ShapeDtypeStruct((n, d_out), jnp.float32),
        scratch_shapes=[pltpu.VMEM((n, d), jnp.bfloat16)],
        compiler_params=pltpu.CompilerParams(
            dimension_semantics=("arbitrary",),
        ),
    )(x, lower, upper, w_irr, w_sol)


def kernel(x, lower_neighborhood, upper_neighborhood, W_irr, W_sol):
    return _run(x, lower_neighborhood, upper_neighborhood, W_irr, W_sol, _BM)


# speedup vs baseline: 1.0054x; 1.0054x over previous
import functools

import jax
import jax.numpy as jnp
from jax.experimental import pallas as pl
from jax.experimental.pallas import tpu as pltpu

_BM = 200


def _body(x_ref, l_ref, u_ref, wi_ref, ws_ref, out_ref, xb_ref):
    m = pl.program_id(0)

    @pl.when(m == 0)
    def _cast_x_once():
        xb_ref[...] = x_ref[...].astype(jnp.bfloat16)

    xb = xb_ref[...]
    lb = l_ref[...].astype(jnp.bfloat16)
    ub = u_ref[...].astype(jnp.bfloat16)
    t_l = jnp.dot(lb, xb, preferred_element_type=jnp.float32)
    t_u = jnp.dot(ub, xb, preferred_element_type=jnp.float32)
    t = (jnp.dot(t_l, wi_ref[...], preferred_element_type=jnp.float32)
         + jnp.dot(t_u, ws_ref[...], preferred_element_type=jnp.float32))
    out_ref[...] = jnp.maximum(t, 0.0)


def _run(x, lower, upper, w_irr, w_sol, bm):
    n, d = x.shape
    d_out = w_irr.shape[1]
    return pl.pallas_call(
        _body,
        grid=(n // bm,),
        in_specs=[
            pl.BlockSpec((n, d), lambda m: (0, 0)),
            pl.BlockSpec((bm, n), lambda m: (m, 0)),
            pl.BlockSpec((bm, n), lambda m: (m, 0)),
            pl.BlockSpec((d, d_out), lambda m: (0, 0)),
            pl.BlockSpec((d, d_out), lambda m: (0, 0)),
        ],
        out_specs=pl.BlockSpec((bm, d_out), lambda m: (m, 0)),
        out_shape=jax.ShapeDtypeStruct((n, d_out), jnp.float32),
        scratch_shapes=[pltpu.VMEM((n, d), jnp.bfloat16)],
        compiler_params=pltpu.CompilerParams(
            dimension_semantics=("parallel",),
        ),
    )(x, lower, upper, w_irr, w_sol)


def kernel(x, lower_neighborhood, upper_neighborhood, W_irr, W_sol):
    return _run(x, lower_neighborhood, upper_neighborhood, W_irr, W_sol, _BM)
